# initial kernel scaffold (unmeasured)
import jax
import jax.numpy as jnp
from jax import lax
from jax.experimental import pallas as pl
from jax.experimental.pallas import tpu as pltpu

N_DEV = 4
CH = 256


def kernel(x, w_mat, scale_x, scale_w):
    M, _ = x.shape
    _, N = w_mat.shape
    n_chunks = M // CH
    R = n_chunks // N_DEV

    def body(x_ref, w_ref, sx_ref, sw_ref, out_ref,
             comm_ref, stage_ref, send_sems, recv_sems, out_sem):
        my = lax.axis_index("i")
        left = (my - 1) % N_DEV
        right = (my + 1) % N_DEV
        scale = sx_ref[0] * sw_ref[0]

        def nbarrier():
            bsem = pltpu.get_barrier_semaphore()
            for nbr in (left, right):
                pl.semaphore_signal(
                    bsem, inc=1,
                    device_id=(nbr,), device_id_type=pl.DeviceIdType.MESH,
                )
            pl.semaphore_wait(bsem, 2)

        def partial(g):
            return lax.dot_general(
                x_ref[pl.ds(g * CH, CH), :], w_ref[...],
                dimension_numbers=(((1,), (0,)), ((), ())),
                preferred_element_type=jnp.float32,
            )

        def hop(h, r):
            rdma = pltpu.make_async_remote_copy(
                src_ref=comm_ref.at[h % 2],
                dst_ref=comm_ref.at[(h + 1) % 2],
                send_sem=send_sems.at[h],
                recv_sem=recv_sems.at[h],
                device_id=(right,),
                device_id_type=pl.DeviceIdType.MESH,
            )
            rdma.start()
            rdma.wait()

        def store(g):
            cp = pltpu.make_async_copy(
                stage_ref, out_ref.at[pl.ds(g * CH, CH), :], out_sem)
            cp.start()
            cp.wait()

        nbarrier()
        for r in range(R):
            if r:
                nbarrier()

            comm_ref[0] = partial(N_DEV * r + my).astype(jnp.bfloat16)
            for h in range(N_DEV - 1):
                hop(h, r)
                r_slot = (h + 1) % 2
                c = (my - h - 1) % N_DEV
                acc = partial(N_DEV * r + c) + comm_ref[r_slot].astype(jnp.float32)
                if h < N_DEV - 2:
                    comm_ref[r_slot] = acc.astype(jnp.bfloat16)
                else:
                    stage_ref[...] = acc * scale
                    comm_ref[r_slot] = stage_ref[...].astype(jnp.bfloat16)

            store(N_DEV * r + (my + 1) % N_DEV)

            for t in range(N_DEV - 1):
                h = (N_DEV - 1) + t
                hop(h, r)
                r_slot = (h + 1) % 2
                stage_ref[...] = comm_ref[r_slot].astype(jnp.float32)
                store(N_DEV * r + (my - t) % N_DEV)

    return pl.pallas_call(
        body,
        out_shape=jax.ShapeDtypeStruct((M, N), jnp.float32),
        in_specs=[
            pl.BlockSpec(memory_space=pltpu.VMEM),
            pl.BlockSpec(memory_space=pltpu.VMEM),
            pl.BlockSpec(memory_space=pltpu.SMEM),
            pl.BlockSpec(memory_space=pltpu.SMEM),
        ],
        out_specs=pl.BlockSpec(memory_space=pltpu.ANY),
        scratch_shapes=[
            pltpu.VMEM((2, CH, N), jnp.bfloat16),
            pltpu.VMEM((CH, N), jnp.float32),
            pltpu.SemaphoreType.DMA((2 * (N_DEV - 1),)),
            pltpu.SemaphoreType.DMA((2 * (N_DEV - 1),)),
            pltpu.SemaphoreType.DMA,
        ],
        compiler_params=pltpu.CompilerParams(collective_id=0),
    )(x, w_mat, scale_x, scale_w)


# baseline (device time: 1340690 ns/iter reference)
import jax
import jax.numpy as jnp
from jax import lax
from jax.experimental import pallas as pl
from jax.experimental.pallas import tpu as pltpu

N_DEV = 4
CH = 256


def kernel(x, w_mat, scale_x, scale_w):
    M, _ = x.shape
    _, N = w_mat.shape
    x = x.astype(jnp.float8_e4m3fn)
    w_mat = w_mat.astype(jnp.float8_e5m2)
    n_chunks = M // CH
    R = n_chunks // N_DEV

    def body(x_ref, w_ref, sx_ref, sw_ref, out_ref,
             comm_ref, stage_ref, send_sems, recv_sems, out_sem):
        my = lax.axis_index("i")
        left = (my - 1) % N_DEV
        right = (my + 1) % N_DEV
        scale = sx_ref[0] * sw_ref[0]

        def nbarrier():
            bsem = pltpu.get_barrier_semaphore()
            for nbr in (left, right):
                pl.semaphore_signal(
                    bsem, inc=1,
                    device_id=(nbr,), device_id_type=pl.DeviceIdType.MESH,
                )
            pl.semaphore_wait(bsem, 2)

        def partial(g):
            return lax.dot_general(
                x_ref[pl.ds(g * CH, CH), :], w_ref[...],
                dimension_numbers=(((1,), (0,)), ((), ())),
                preferred_element_type=jnp.float32,
            )

        def hop(h, r):
            rdma = pltpu.make_async_remote_copy(
                src_ref=comm_ref.at[h % 2],
                dst_ref=comm_ref.at[(h + 1) % 2],
                send_sem=send_sems.at[h],
                recv_sem=recv_sems.at[h],
                device_id=(right,),
                device_id_type=pl.DeviceIdType.MESH,
            )
            rdma.start()
            rdma.wait()

        def store(g):
            cp = pltpu.make_async_copy(
                stage_ref, out_ref.at[pl.ds(g * CH, CH), :], out_sem)
            cp.start()
            cp.wait()

        nbarrier()
        for r in range(R):
            if r:
                nbarrier()

            comm_ref[0] = partial(N_DEV * r + my).astype(jnp.bfloat16)
            for h in range(N_DEV - 1):
                hop(h, r)
                r_slot = (h + 1) % 2
                c = (my - h - 1) % N_DEV
                acc = partial(N_DEV * r + c) + comm_ref[r_slot].astype(jnp.float32)
                if h < N_DEV - 2:
                    comm_ref[r_slot] = acc.astype(jnp.bfloat16)
                else:
                    stage_ref[...] = acc * scale
                    comm_ref[r_slot] = stage_ref[...].astype(jnp.bfloat16)

            store(N_DEV * r + (my + 1) % N_DEV)

            for t in range(N_DEV - 1):
                h = (N_DEV - 1) + t
                hop(h, r)
                r_slot = (h + 1) % 2
                stage_ref[...] = comm_ref[r_slot].astype(jnp.float32)
                store(N_DEV * r + (my - t) % N_DEV)

    return pl.pallas_call(
        body,
        out_shape=jax.ShapeDtypeStruct((M, N), jnp.float32),
        in_specs=[
            pl.BlockSpec(memory_space=pltpu.VMEM),
            pl.BlockSpec(memory_space=pltpu.VMEM),
            pl.BlockSpec(memory_space=pltpu.SMEM),
            pl.BlockSpec(memory_space=pltpu.SMEM),
        ],
        out_specs=pl.BlockSpec(memory_space=pl.ANY),
        scratch_shapes=[
            pltpu.VMEM((2, CH, N), jnp.bfloat16),
            pltpu.VMEM((CH, N), jnp.float32),
            pltpu.SemaphoreType.DMA((2 * (N_DEV - 1),)),
            pltpu.SemaphoreType.DMA((2 * (N_DEV - 1),)),
            pltpu.SemaphoreType.DMA,
        ],
        compiler_params=pltpu.CompilerParams(
            collective_id=0,
            vmem_limit_bytes=60 * 1024 * 1024,
        ),
    )(x, w_mat, scale_x, scale_w)


# device time: 717193 ns/iter; 1.8694x vs baseline; 1.8694x over previous
import jax
import jax.numpy as jnp
from jax import lax
from jax.experimental import pallas as pl
from jax.experimental.pallas import tpu as pltpu

N_DEV = 4
CH = 256
CW, CCW = 0, 1


def kernel(x, w_mat, scale_x, scale_w):
    M, _ = x.shape
    _, N = w_mat.shape
    x = x.astype(jnp.float8_e4m3fn)
    w_mat = w_mat.astype(jnp.float8_e5m2)

    n_chunks = M // CH
    P = n_chunks // (2 * N_DEV)

    def body(x_ref, w_ref, sx_ref, sw_ref, out_ref,
             comm_ref, stage_ref, send_sems, recv_sems, out_sems):
        my = lax.axis_index("i")
        left = (my - 1) % N_DEV
        right = (my + 1) % N_DEV
        nbr = {CW: right, CCW: left}
        scale = sx_ref[0] * sw_ref[0]

        def nbarrier():
            bsem = pltpu.get_barrier_semaphore()
            for t in (left, right):
                pl.semaphore_signal(
                    bsem, inc=1,
                    device_id=(t,), device_id_type=pl.DeviceIdType.MESH,
                )
            pl.semaphore_wait(bsem, 2)

        def partial(g):
            return lax.dot_general(
                x_ref[pl.ds(g * CH, CH), :], w_ref[...],
                dimension_numbers=(((1,), (0,)), ((), ())),
                preferred_element_type=jnp.float32,
            )

        def ring_rdma(d, h):
            return pltpu.make_async_remote_copy(
                src_ref=comm_ref.at[d, h % 2],
                dst_ref=comm_ref.at[d, (h + 1) % 2],
                send_sem=send_sems.at[d, h],
                recv_sem=recv_sems.at[d, h],
                device_id=(nbr[d],),
                device_id_type=pl.DeviceIdType.MESH,
            )

        pending = {CW: None, CCW: None}

        def store(d, g):
            cp = pltpu.make_async_copy(
                stage_ref.at[d], out_ref.at[pl.ds(g * CH, CH), :],
                out_sems.at[d])
            cp.start()
            pending[d] = cp

        def wait_store(d):
            if pending[d] is not None:
                pending[d].wait()
                pending[d] = None

        def g_of(p, d, c):
            base = 2 * N_DEV * p + N_DEV * d
            return base + c

        def rs_recv_c(d, h):
            return (my - h - 1) % N_DEV if d == CW else (my + h + 1) % N_DEV

        def own_c(d):
            return (my + 1) % N_DEV if d == CW else (my - 1) % N_DEV

        def ag_recv_c(d, t):
            return (my - t) % N_DEV if d == CW else (my + t) % N_DEV

        nbarrier()
        for p in range(P):
            if p:
                nbarrier()

            for d in (CW, CCW):
                comm_ref[d, 0] = partial(g_of(p, d, my)).astype(jnp.bfloat16)
            for h in range(N_DEV - 1):
                rdmas = [ring_rdma(d, h) for d in (CW, CCW)]
                for r in rdmas:
                    r.start()
                for d in (CW, CCW):
                    stage_ref[d] = partial(g_of(p, d, rs_recv_c(d, h)))
                for r in rdmas:
                    r.wait()
                r_slot = (h + 1) % 2
                for d in (CW, CCW):
                    if h < N_DEV - 2:
                        comm_ref[d, r_slot] = (
                            stage_ref[d][...]
                            + comm_ref[d, r_slot].astype(jnp.float32)
                        ).astype(jnp.bfloat16)
                    else:
                        stage_ref[d] = (
                            stage_ref[d][...]
                            + comm_ref[d, r_slot].astype(jnp.float32)
                        ) * scale
                        comm_ref[d, r_slot] = (
                            stage_ref[d][...].astype(jnp.bfloat16))

            ag = [[ring_rdma(d, (N_DEV - 1) + t) for d in (CW, CCW)]
                  for t in range(N_DEV - 1)]
            for r in ag[0]:
                r.start()
            for d in (CW, CCW):
                store(d, g_of(p, d, own_c(d)))
            for t in range(N_DEV - 1):
                for r in ag[t]:
                    r.wait()
                if t < N_DEV - 2:
                    for r in ag[t + 1]:
                        r.start()
                r_slot = (N_DEV + t) % 2
                for d in (CW, CCW):
                    wait_store(d)
                    stage_ref[d] = comm_ref[d, r_slot].astype(jnp.float32)
                    store(d, g_of(p, d, ag_recv_c(d, t)))
            for d in (CW, CCW):
                wait_store(d)

    return pl.pallas_call(
        body,
        out_shape=jax.ShapeDtypeStruct((M, N), jnp.float32),
        in_specs=[
            pl.BlockSpec(memory_space=pltpu.VMEM),
            pl.BlockSpec(memory_space=pltpu.VMEM),
            pl.BlockSpec(memory_space=pltpu.SMEM),
            pl.BlockSpec(memory_space=pltpu.SMEM),
        ],
        out_specs=pl.BlockSpec(memory_space=pl.ANY),
        scratch_shapes=[
            pltpu.VMEM((2, 2, CH, N), jnp.bfloat16),
            pltpu.VMEM((2, CH, N), jnp.float32),
            pltpu.SemaphoreType.DMA((2, 2 * (N_DEV - 1))),
            pltpu.SemaphoreType.DMA((2, 2 * (N_DEV - 1))),
            pltpu.SemaphoreType.DMA((2,)),
        ],
        compiler_params=pltpu.CompilerParams(
            collective_id=0,
            vmem_limit_bytes=60 * 1024 * 1024,
        ),
    )(x, w_mat, scale_x, scale_w)


# device time: 708707 ns/iter; 1.8917x vs baseline; 1.0120x over previous
import jax
import jax.numpy as jnp
from jax import lax
from jax.experimental import pallas as pl
from jax.experimental.pallas import tpu as pltpu

N_DEV = 4
CH = 256
CW, CCW = 0, 1


def kernel(x, w_mat, scale_x, scale_w):
    M, _ = x.shape
    _, N = w_mat.shape
    x = x.astype(jnp.float8_e4m3fn)
    w_mat = w_mat.astype(jnp.float8_e5m2)

    n_chunks = M // CH
    P = n_chunks // (2 * N_DEV)

    def body(x_ref, w_ref, sx_ref, sw_ref, out_ref,
             comm_ref, stage_ref, send_sems, recv_sems, out_sems):
        my = lax.axis_index("i")
        left = (my - 1) % N_DEV
        right = (my + 1) % N_DEV
        nbr = {CW: right, CCW: left}
        scale = sx_ref[0] * sw_ref[0]

        def nbarrier():
            bsem = pltpu.get_barrier_semaphore()
            for t in (left, right):
                pl.semaphore_signal(
                    bsem, inc=1,
                    device_id=(t,), device_id_type=pl.DeviceIdType.MESH,
                )
            pl.semaphore_wait(bsem, 2)

        def partial(g):
            return lax.dot_general(
                x_ref[pl.ds(g * CH, CH), :], w_ref[...],
                dimension_numbers=(((1,), (0,)), ((), ())),
                preferred_element_type=jnp.float32,
            )

        def ring_rdma(d, h):
            return pltpu.make_async_remote_copy(
                src_ref=comm_ref.at[d, h % 2],
                dst_ref=comm_ref.at[d, (h + 1) % 2],
                send_sem=send_sems.at[d, h],
                recv_sem=recv_sems.at[d, h],
                device_id=(nbr[d],),
                device_id_type=pl.DeviceIdType.MESH,
            )

        pending = {CW: None, CCW: None}

        def store(d, g):
            cp = pltpu.make_async_copy(
                stage_ref.at[d], out_ref.at[pl.ds(g * CH, CH), :],
                out_sems.at[d])
            cp.start()
            pending[d] = cp

        def wait_store(d):
            if pending[d] is not None:
                pending[d].wait()
                pending[d] = None

        def g_of(p, d, c):
            base = 2 * N_DEV * p + N_DEV * d
            return base + c

        def rs_recv_c(d, h):
            return (my - h - 1) % N_DEV if d == CW else (my + h + 1) % N_DEV

        def own_c(d):
            return (my + 1) % N_DEV if d == CW else (my - 1) % N_DEV

        def ag_recv_c(d, t):
            return (my - t) % N_DEV if d == CW else (my + t) % N_DEV

        nbarrier()
        for p in range(P):
            if p:
                nbarrier()

            rdmas = {}
            for d in (CW, CCW):
                comm_ref[d, 0] = partial(g_of(p, d, my)).astype(jnp.bfloat16)
                rdmas[d] = ring_rdma(d, 0)
                rdmas[d].start()
            for h in range(N_DEV - 1):
                for d in (CW, CCW):
                    wait_store(d)
                    stage_ref[d] = partial(g_of(p, d, rs_recv_c(d, h)))
                r_slot = (h + 1) % 2
                for d in (CW, CCW):
                    rdmas[d].wait()
                    if h < N_DEV - 2:
                        comm_ref[d, r_slot] = (
                            stage_ref[d][...]
                            + comm_ref[d, r_slot].astype(jnp.float32)
                        ).astype(jnp.bfloat16)
                        rdmas[d] = ring_rdma(d, h + 1)
                        rdmas[d].start()
                    else:
                        stage_ref[d] = (
                            stage_ref[d][...]
                            + comm_ref[d, r_slot].astype(jnp.float32)
                        ) * scale
                        comm_ref[d, r_slot] = (
                            stage_ref[d][...].astype(jnp.bfloat16))
                        rdmas[d] = ring_rdma(d, N_DEV - 1)
                        rdmas[d].start()
                        store(d, g_of(p, d, own_c(d)))

            for t in range(N_DEV - 1):
                r_slot = (N_DEV + t) % 2
                for d in (CW, CCW):
                    rdmas[d].wait()
                    if t < N_DEV - 2:
                        rdmas[d] = ring_rdma(d, N_DEV + t)
                        rdmas[d].start()
                for d in (CW, CCW):
                    wait_store(d)
                    stage_ref[d] = comm_ref[d, r_slot].astype(jnp.float32)
                    store(d, g_of(p, d, ag_recv_c(d, t)))
            for d in (CW, CCW):
                wait_store(d)

    return pl.pallas_call(
        body,
        out_shape=jax.ShapeDtypeStruct((M, N), jnp.float32),
        in_specs=[
            pl.BlockSpec(memory_space=pltpu.VMEM),
            pl.BlockSpec(memory_space=pltpu.VMEM),
            pl.BlockSpec(memory_space=pltpu.SMEM),
            pl.BlockSpec(memory_space=pltpu.SMEM),
        ],
        out_specs=pl.BlockSpec(memory_space=pl.ANY),
        scratch_shapes=[
            pltpu.VMEM((2, 2, CH, N), jnp.bfloat16),
            pltpu.VMEM((2, CH, N), jnp.float32),
            pltpu.SemaphoreType.DMA((2, 2 * (N_DEV - 1))),
            pltpu.SemaphoreType.DMA((2, 2 * (N_DEV - 1))),
            pltpu.SemaphoreType.DMA((2,)),
        ],
        compiler_params=pltpu.CompilerParams(
            collective_id=0,
            vmem_limit_bytes=60 * 1024 * 1024,
        ),
    )(x, w_mat, scale_x, scale_w)
